# baseline (speedup 1.0000x reference)
import functools

import jax
import jax.numpy as jnp
from jax import lax
from jax.experimental import pallas as pl
from jax.experimental.pallas import tpu as pltpu
from jax.experimental.pallas import tpu_sc as plsc

_NUM_CORES = 2
_NUM_SUBCORES = 16
_NUM_WORKERS = _NUM_CORES * _NUM_SUBCORES
_LANES = 16


@functools.cache
def _build_gather(B: int, S: int, V: int, D: int, Vpad: int):
    cols = B // _NUM_WORKERS
    shi_n = S // 8
    rows = 8 * D
    groups = cols // _LANES
    assert B % (_NUM_WORKERS * 128) == 0 and S % 8 == 0
    assert shi_n % 2 == 1 and shi_n >= 3

    mesh = plsc.VectorSubcoreMesh(core_axis_name="c", subcore_axis_name="s")

    @functools.partial(
        pl.kernel,
        mesh=mesh,
        compiler_params=pltpu.CompilerParams(needs_layout_passes=False),
        out_type=jax.ShapeDtypeStruct((S * D, B), jnp.float32),
        scratch_types=[
            pltpu.VMEM((Vpad * D * _LANES,), jnp.float32),
            pltpu.VMEM((8, cols), jnp.int32),
            pltpu.VMEM((8, cols), jnp.int32),
            pltpu.VMEM((rows, cols), jnp.float32),
            pltpu.VMEM((rows, cols), jnp.float32),
            pltpu.SemaphoreType.DMA,
            pltpu.SemaphoreType.DMA,
            pltpu.SemaphoreType.DMA,
            pltpu.SemaphoreType.DMA,
        ],
    )
    def gather_kernel(table_hbm, idx_hbm, out_hbm, table_v, idx_0, idx_1,
                      stage_0, stage_1, sem_i0, sem_i1, sem_o0, sem_o1):
        wid = lax.axis_index("s") * _NUM_CORES + lax.axis_index("c")
        col0 = wid * cols

        pltpu.sync_copy(table_hbm, table_v)

        def start_idx(shi, idx_v, sem):
            pltpu.async_copy(
                idx_hbm.at[pl.ds(shi * 8, 8), pl.ds(col0, cols)], idx_v, sem)

        def wait_idx(idx_v, sem):
            pltpu.make_async_copy(
                idx_hbm.at[pl.ds(0, 8), pl.ds(0, cols)], idx_v, sem).wait()

        def start_out(shi, stage_v, sem):
            pltpu.async_copy(
                stage_v, out_hbm.at[pl.ds(shi * rows, rows),
                                    pl.ds(col0, cols)], sem)

        def wait_out(stage_v, sem):
            pltpu.make_async_copy(
                stage_v, out_hbm.at[pl.ds(0, rows), pl.ds(0, cols)],
                sem).wait()

        iota = lax.broadcasted_iota(jnp.int32, (_LANES,), 0)

        def compute(idx_v, stage_v):
            @plsc.parallel_loop(0, groups, 1, unroll=4)
            def gbody(g):
                for sl in range(8):
                    v = idx_v[sl, pl.ds(g * _LANES, _LANES)]
                    u = v * _LANES + iota
                    for d in range(D):
                        gv = plsc.load_gather(
                            table_v.at[pl.ds(d * Vpad * _LANES, Vpad * _LANES)],
                            [u])
                        stage_v[sl * D + d, pl.ds(g * _LANES, _LANES)] = gv

        start_idx(0, idx_0, sem_i0)

        wait_idx(idx_0, sem_i0)
        start_idx(1, idx_1, sem_i1)
        compute(idx_0, stage_0)
        start_out(0, stage_0, sem_o0)

        def body(gp, carry):
            i1 = gp * 2 + 1
            wait_idx(idx_1, sem_i1)

            @pl.when(i1 + 1 < shi_n)
            def _():
                start_idx(i1 + 1, idx_0, sem_i0)

            @pl.when(gp >= 1)
            def _():
                wait_out(stage_1, sem_o1)

            compute(idx_1, stage_1)
            start_out(i1, stage_1, sem_o1)

            i0 = gp * 2 + 2
            wait_idx(idx_0, sem_i0)

            @pl.when(i0 + 1 < shi_n)
            def _():
                start_idx(i0 + 1, idx_1, sem_i1)

            wait_out(stage_0, sem_o0)
            compute(idx_0, stage_0)
            start_out(i0, stage_0, sem_o0)
            return carry

        lax.fori_loop(0, (shi_n - 1) // 2, body, 0)
        wait_out(stage_0, sem_o0)
        wait_out(stage_1, sem_o1)

    return gather_kernel


def kernel(koppen_codes, embedding_table):
    B0, S = koppen_codes.shape
    V, D = embedding_table.shape
    idx2d = koppen_codes.astype(jnp.int32).T
    Vpad = (V + 7) // 8 * 8
    table_rep = jnp.repeat(
        jnp.pad(embedding_table, ((0, Vpad - V), (0, 0))).T.reshape(-1),
        _LANES)
    o2 = _build_gather(B0, S, V, D, Vpad)(table_rep, idx2d)
    return o2.reshape(S, D, B0).transpose(2, 0, 1)


# speedup vs baseline: 1.2932x; 1.2932x over previous
import functools

import jax
import jax.numpy as jnp
from jax import lax
from jax.experimental import pallas as pl
from jax.experimental.pallas import tpu as pltpu
from jax.experimental.pallas import tpu_sc as plsc

_NUM_CORES = 2
_NUM_SUBCORES = 16
_NUM_WORKERS = _NUM_CORES * _NUM_SUBCORES
_LANES = 16


@functools.cache
def _build_gather(B: int, S: int, V: int, D: int, Vpad: int):
    cols = B // _NUM_WORKERS
    shi_n = S // 8
    rows = 8 * D
    groups = cols // _LANES
    assert B % (_NUM_WORKERS * 128) == 0 and S % 8 == 0
    assert shi_n % 2 == 1 and shi_n >= 3

    mesh = plsc.VectorSubcoreMesh(core_axis_name="c", subcore_axis_name="s")

    @functools.partial(
        pl.kernel,
        mesh=mesh,
        compiler_params=pltpu.CompilerParams(needs_layout_passes=False),
        out_type=jax.ShapeDtypeStruct((S * D, B), jnp.float32),
        scratch_types=[
            pltpu.VMEM((Vpad * D * _LANES,), jnp.float32),
            pltpu.VMEM((8, cols), jnp.int32),
            pltpu.VMEM((8, cols), jnp.int32),
            pltpu.VMEM((rows, cols), jnp.float32),
            pltpu.VMEM((rows, cols), jnp.float32),
            pltpu.SemaphoreType.DMA,
            pltpu.SemaphoreType.DMA,
            pltpu.SemaphoreType.DMA,
            pltpu.SemaphoreType.DMA,
        ],
    )
    def gather_kernel(table_hbm, idx_hbm, out_hbm, table_v, idx_0, idx_1,
                      stage_0, stage_1, sem_i0, sem_i1, sem_o0, sem_o1):
        wid = lax.axis_index("s") * _NUM_CORES + lax.axis_index("c")
        col0 = wid * cols

        pltpu.sync_copy(table_hbm, table_v)

        def start_idx(shi, idx_v, sem):
            pltpu.async_copy(
                idx_hbm.at[pl.ds(shi * 8, 8), pl.ds(col0, cols)], idx_v, sem)

        def wait_idx(idx_v, sem):
            pltpu.make_async_copy(
                idx_hbm.at[pl.ds(0, 8), pl.ds(0, cols)], idx_v, sem).wait()

        def start_out(shi, stage_v, sem):
            pltpu.async_copy(
                stage_v, out_hbm.at[pl.ds(shi * rows, rows),
                                    pl.ds(col0, cols)], sem)

        def wait_out(stage_v, sem):
            pltpu.make_async_copy(
                stage_v, out_hbm.at[pl.ds(0, rows), pl.ds(0, cols)],
                sem).wait()

        iota = lax.broadcasted_iota(jnp.int32, (_LANES,), 0)

        def compute(idx_v, stage_v):
            @plsc.parallel_loop(0, groups, 1, unroll=8)
            def gbody(g):
                for sl in range(8):
                    v = idx_v[sl, pl.ds(g * _LANES, _LANES)]
                    u = v * _LANES + iota
                    for d in range(D):
                        gv = plsc.load_gather(
                            table_v.at[pl.ds(d * Vpad * _LANES, Vpad * _LANES)],
                            [u])
                        stage_v[sl * D + d, pl.ds(g * _LANES, _LANES)] = gv

        start_idx(0, idx_0, sem_i0)

        wait_idx(idx_0, sem_i0)
        start_idx(1, idx_1, sem_i1)
        compute(idx_0, stage_0)
        start_out(0, stage_0, sem_o0)

        def body(gp, carry):
            i1 = gp * 2 + 1
            wait_idx(idx_1, sem_i1)

            @pl.when(i1 + 1 < shi_n)
            def _():
                start_idx(i1 + 1, idx_0, sem_i0)

            @pl.when(gp >= 1)
            def _():
                wait_out(stage_1, sem_o1)

            compute(idx_1, stage_1)
            start_out(i1, stage_1, sem_o1)

            i0 = gp * 2 + 2
            wait_idx(idx_0, sem_i0)

            @pl.when(i0 + 1 < shi_n)
            def _():
                start_idx(i0 + 1, idx_1, sem_i1)

            wait_out(stage_0, sem_o0)
            compute(idx_0, stage_0)
            start_out(i0, stage_0, sem_o0)
            return carry

        lax.fori_loop(0, (shi_n - 1) // 2, body, 0)
        wait_out(stage_0, sem_o0)
        wait_out(stage_1, sem_o1)

    return gather_kernel


def kernel(koppen_codes, embedding_table):
    B0, S = koppen_codes.shape
    V, D = embedding_table.shape
    idx2d = koppen_codes.astype(jnp.int32).T
    Vpad = (V + 7) // 8 * 8
    table_rep = jnp.repeat(
        jnp.pad(embedding_table, ((0, Vpad - V), (0, 0))).T.reshape(-1),
        _LANES)
    o2 = _build_gather(B0, S, V, D, Vpad)(table_rep, idx2d)
    return o2.reshape(S, D, B0).transpose(2, 0, 1)
